# Initial kernel scaffold; baseline (speedup 1.0000x reference)
import jax
import jax.numpy as jnp
from jax.experimental import pallas as pl


def kernel(quantized):
    raise NotImplementedError("write your pallas kernel here")


# speedup vs baseline: 3.7255x; 3.7255x over previous
import jax
import jax.numpy as jnp
from jax.experimental import pallas as pl
from jax.experimental.pallas import tpu as pltpu

_PROBABILITY = 0.12
_T = 2048
_ROWS_PER_BLOCK = 512


def _selectors():
    key = jax.random.key(42)
    k_replace, k_dir = jax.random.split(key)
    replace = jax.random.uniform(k_replace, (_T,)) < _PROBABILITY
    direction = jnp.where(jax.random.uniform(k_dir, (_T,)) < 0.5, -1, 1)
    pos = jnp.arange(_T)
    neighbor = jnp.where(
        pos == 0, 1, jnp.where(pos == _T - 1, _T - 2, pos + direction)
    )
    take_left = replace & (neighbor > pos)
    take_right = replace & (neighbor < pos)
    return take_left, take_right


def _jitter_body(x_ref, left_ref, right_ref, o_ref):
    x = x_ref[...]
    left = left_ref[...]
    right = right_ref[...]
    shift_left = pltpu.roll(x, _T - 1, axis=1)
    shift_right = pltpu.roll(x, 1, axis=1)
    o_ref[...] = jnp.where(left, shift_left, jnp.where(right, shift_right, x))


def kernel(quantized):
    B, C, T = quantized.shape
    rows = B * C
    x2d = quantized.reshape(rows, T)
    take_left, take_right = _selectors()
    left2d = take_left.reshape(1, T)
    right2d = take_right.reshape(1, T)
    grid = rows // _ROWS_PER_BLOCK
    out = pl.pallas_call(
        _jitter_body,
        grid=(grid,),
        in_specs=[
            pl.BlockSpec((_ROWS_PER_BLOCK, T), lambda i: (i, 0)),
            pl.BlockSpec((1, T), lambda i: (0, 0)),
            pl.BlockSpec((1, T), lambda i: (0, 0)),
        ],
        out_specs=pl.BlockSpec((_ROWS_PER_BLOCK, T), lambda i: (i, 0)),
        out_shape=jax.ShapeDtypeStruct((rows, T), jnp.float32),
    )(x2d, left2d, right2d)
    return out.reshape(B, C, T)
